# Initial kernel scaffold; baseline (speedup 1.0000x reference)
import jax
import jax.numpy as jnp
from jax.experimental import pallas as pl


def kernel(indices, offsets, edge_index, ppi_w, self_w, emb_table, input_bias, W1, b1, W2, b2, Wout, bout):
    raise NotImplementedError("write your pallas kernel here")


# speedup vs baseline: 7.5553x; 7.5553x over previous
import functools

import jax
import jax.numpy as jnp
from jax import lax
from jax.experimental import pallas as pl
from jax.experimental.pallas import tpu as pltpu
from jax.experimental.pallas import tpu_sc as plsc

_N = 10000
_E = 320000
_H = 128
_L = 121
_NNZ = 50
_NC = 2
_NS = 16
_NW = _NC * _NS

_GROUP_NODES = 8
_GROUP_ROWS = _GROUP_NODES * _NNZ
_NGROUPS = _N // _GROUP_NODES

_mesh = plsc.VectorSubcoreMesh(
    core_axis_name="c", subcore_axis_name="s", num_cores=_NC, num_subcores=_NS
)


@functools.partial(
    pl.kernel,
    out_type=jax.ShapeDtypeStruct((_N, _H), jnp.float32),
    mesh=_mesh,
    scratch_types=[
        pltpu.VMEM((_GROUP_ROWS,), jnp.int32),
        pltpu.VMEM((_GROUP_ROWS, _H), jnp.float32),
        pltpu.VMEM((_H,), jnp.float32),
        pltpu.VMEM((_GROUP_NODES, _H), jnp.float32),
        pltpu.SemaphoreType.DMA,
    ],
)
def _embed_sc(idx_hbm, tab_hbm, bias_hbm, out_hbm, idx_v, rows_v, bias_v, out_v, sem):
    wid = lax.axis_index("s") * _NC + lax.axis_index("c")
    pltpu.sync_copy(bias_hbm, bias_v)
    bias = [bias_v[pl.ds(16 * j, 16)] for j in range(_H // 16)]
    ngroups = (_NGROUPS - 1 - wid) // _NW + 1

    def group_body(i, carry):
        g = wid + i * _NW
        pltpu.sync_copy(idx_hbm.at[pl.ds(g * _GROUP_ROWS, _GROUP_ROWS)], idx_v)
        copies = [
            pltpu.async_copy(
                tab_hbm.at[idx_v[pl.ds(k * 16, 16)]],
                rows_v.at[pl.ds(k * 16, 16), :],
                sem,
            )
            for k in range(_GROUP_ROWS // 16)
        ]
        for cp in copies:
            cp.wait()
        for n in range(_GROUP_NODES):
            acc = tuple(rows_v[n * _NNZ, pl.ds(16 * j, 16)] for j in range(_H // 16))

            def add_rows(m, a):
                base = n * _NNZ + 1 + m * 7
                for t in range(7):
                    a = tuple(
                        a[j] + rows_v[base + t, pl.ds(16 * j, 16)]
                        for j in range(_H // 16)
                    )
                return a

            acc = lax.fori_loop(0, 7, add_rows, acc)
            for j in range(_H // 16):
                out_v[n, pl.ds(16 * j, 16)] = jnp.maximum(acc[j] + bias[j], 0.0)
        pltpu.sync_copy(out_v, out_hbm.at[pl.ds(g * _GROUP_NODES, _GROUP_NODES), :])
        return carry

    lax.fori_loop(0, ngroups, group_body, 0)


_ECH = 256
_NECH = _E // _ECH
_ZCH = 80
_NZCH = _N // _ZCH


@functools.partial(
    pl.kernel,
    out_type=jax.ShapeDtypeStruct((_NC, _N, _H), jnp.float32),
    mesh=_mesh,
    scratch_types=[
        pltpu.VMEM((_ECH,), jnp.int32),
        pltpu.VMEM((_ECH,), jnp.int32),
        pltpu.VMEM((_ECH,), jnp.float32),
        pltpu.VMEM((_ECH, _H), jnp.float32),
        pltpu.SemaphoreType.DMA,
        pltpu.VMEM_SHARED((_N, _H), jnp.float32),
    ],
)
def _edges_sc(h_hbm, src_hbm, dst_hbm, w_hbm, z_hbm, out_hbm,
              src_v, dst_v, w_v, rows_v, sem, acc):
    cid = lax.axis_index("c")
    sid = lax.axis_index("s")
    nz = (_NZCH - 1 - sid) // _NS + 1

    def zero_body(i, carry):
        r0 = (sid + i * _NS) * _ZCH
        pltpu.sync_copy(z_hbm.at[pl.ds(r0, _ZCH), :], acc.at[pl.ds(r0, _ZCH), :])
        return carry

    lax.fori_loop(0, nz, zero_body, 0)
    plsc.subcore_barrier()

    ne = (_NECH - 1 - sid) // _NS + 1

    def chunk_body(c, carry):
        e0 = (sid + c * _NS) * _ECH
        pltpu.sync_copy(src_hbm.at[pl.ds(e0, _ECH)], src_v)
        pltpu.sync_copy(dst_hbm.at[pl.ds(e0, _ECH)], dst_v)
        pltpu.sync_copy(w_hbm.at[pl.ds(cid * _E + e0, _ECH)], w_v)
        copies = [
            pltpu.async_copy(
                h_hbm.at[src_v[pl.ds(k * 16, 16)]],
                rows_v.at[pl.ds(k * 16, 16), :],
                sem,
            )
            for k in range(_ECH // 16)
        ]
        for cp in copies:
            cp.wait()

        def scale_body(k, carry2):
            kb = k * 16
            w16 = w_v[pl.ds(kb, 16)]
            for e in range(16):
                r = kb + e
                wb = lax.gather(
                    w16, jnp.full((16, 1), e, jnp.int32),
                    lax.GatherDimensionNumbers(
                        offset_dims=(), collapsed_slice_dims=(0,),
                        start_index_map=(0,)),
                    (1,), mode=lax.GatherScatterMode.PROMISE_IN_BOUNDS)
                for j in range(_H // 16):
                    rows_v[r, pl.ds(16 * j, 16)] = rows_v[r, pl.ds(16 * j, 16)] * wb
            return carry2

        lax.fori_loop(0, _ECH // 16, scale_body, 0)
        for k in range(_ECH // 16):
            pltpu.sync_copy(rows_v.at[pl.ds(k * 16, 16), :],
                            acc.at[dst_v[pl.ds(k * 16, 16)]], add=True)
        return carry

    lax.fori_loop(0, ne, chunk_body, 0)
    plsc.subcore_barrier()

    def wb_body(i, carry):
        r0 = (sid + i * _NS) * _ZCH
        pltpu.sync_copy(acc.at[pl.ds(r0, _ZCH), :],
                        out_hbm.at[cid, pl.ds(r0, _ZCH), :])
        return carry

    lax.fori_loop(0, nz, wb_body, 0)


_BLK = 1000


def _upd_body(p_ref, r_ref, w_ref, b_ref, o_ref):
    o_ref[...] = (
        jnp.maximum(
            jnp.dot(p_ref[...], w_ref[...], preferred_element_type=jnp.float32)
            + b_ref[...],
            0.0,
        )
        + r_ref[...]
    )


def _update_tc(ppi, res, W, b):
    return pl.pallas_call(
        _upd_body,
        grid=(_N // _BLK,),
        in_specs=[
            pl.BlockSpec((_BLK, _H), lambda i: (i, 0)),
            pl.BlockSpec((_BLK, _H), lambda i: (i, 0)),
            pl.BlockSpec((_H, _H), lambda i: (0, 0)),
            pl.BlockSpec((1, _H), lambda i: (0, 0)),
        ],
        out_specs=pl.BlockSpec((_BLK, _H), lambda i: (i, 0)),
        out_shape=jax.ShapeDtypeStruct((_N, _H), jnp.float32),
    )(ppi, res, W, b.reshape(1, _H))


def _fin_body(h_ref, w_ref, b_ref, o_ref):
    o_ref[...] = (
        jnp.dot(h_ref[...], w_ref[...], preferred_element_type=jnp.float32)
        + b_ref[...]
    )


def _final_tc(h, Wp, bp):
    return pl.pallas_call(
        _fin_body,
        grid=(_N // _BLK,),
        in_specs=[
            pl.BlockSpec((_BLK, _H), lambda i: (i, 0)),
            pl.BlockSpec((_H, _H), lambda i: (0, 0)),
            pl.BlockSpec((1, _H), lambda i: (0, 0)),
        ],
        out_specs=pl.BlockSpec((_BLK, _H), lambda i: (i, 0)),
        out_shape=jax.ShapeDtypeStruct((_N, _H), jnp.float32),
    )(h, Wp, bp.reshape(1, _H))


def kernel(indices, offsets, edge_index, ppi_w, self_w,
           emb_table, input_bias, W1, b1, W2, b2, Wout, bout):
    del offsets
    src = edge_index[0]
    dst = edge_index[1]
    w2 = jnp.concatenate([self_w, ppi_w])
    zeros = jnp.zeros((_N, _H), jnp.float32)
    h = _embed_sc(indices, emb_table, input_bias)
    for (W, b) in ((W1, b1), (W2, b2)):
        outs = _edges_sc(h, src, dst, w2, zeros)
        h = _update_tc(outs[1], outs[0], W, b)
    Wp = jnp.pad(Wout, ((0, 0), (0, _H - _L)))
    bp = jnp.pad(bout, (0, _H - _L))
    return _final_tc(h, Wp, bp)[:, :_L]


# speedup vs baseline: 15.4513x; 2.0451x over previous
import functools

import jax
import jax.numpy as jnp
from jax import lax
from jax.experimental import pallas as pl
from jax.experimental.pallas import tpu as pltpu
from jax.experimental.pallas import tpu_sc as plsc

_N = 10000
_E = 320000
_H = 128
_L = 121
_NNZ = 50
_NC = 2
_NS = 16
_NW = _NC * _NS

_mesh = plsc.VectorSubcoreMesh(
    core_axis_name="c", subcore_axis_name="s", num_cores=_NC, num_subcores=_NS
)


def _bcast_lane(v16, e):
    return lax.gather(
        v16, jnp.full((16, 1), e, jnp.int32),
        lax.GatherDimensionNumbers(
            offset_dims=(), collapsed_slice_dims=(0,), start_index_map=(0,)),
        (1,), mode=lax.GatherScatterMode.PROMISE_IN_BOUNDS)


_GN = 8
_GR = _GN * _NNZ
_NG = _N // _GN


@functools.partial(
    pl.kernel,
    out_type=jax.ShapeDtypeStruct((_N, _H), jnp.float32),
    mesh=_mesh,
    scratch_types=[
        pltpu.VMEM((_GR,), jnp.int32),
        pltpu.VMEM((_GR,), jnp.int32),
        pltpu.VMEM((_GR, _H), jnp.float32),
        pltpu.VMEM((_GR, _H), jnp.float32),
        pltpu.VMEM((_H,), jnp.float32),
        pltpu.VMEM((_GN, _H), jnp.float32),
        pltpu.VMEM((_GN, _H), jnp.float32),
        pltpu.SemaphoreType.DMA,
        pltpu.SemaphoreType.DMA,
        pltpu.SemaphoreType.DMA,
        pltpu.SemaphoreType.DMA,
        pltpu.SemaphoreType.DMA,
        pltpu.SemaphoreType.DMA,
    ],
)
def _embed_sc(idx_hbm, tab_hbm, bias_hbm, out_hbm,
              idx0, idx1, rows0, rows1, bias_v, out0, out1,
              isem0, isem1, gsem0, gsem1, osem0, osem1):
    wid = lax.axis_index("s") * _NC + lax.axis_index("c")
    idx_b = (idx0, idx1)
    rows_b = (rows0, rows1)
    out_b = (out0, out1)
    isem = (isem0, isem1)
    gsem = (gsem0, gsem1)
    osem = (osem0, osem1)
    pltpu.sync_copy(bias_hbm, bias_v)
    bias = [bias_v[pl.ds(16 * j, 16)] for j in range(_H // 16)]
    ng = (_NG - 1 - wid) // _NW + 1

    def stage(i, b):
        @pl.when(i < ng)
        def _():
            g0 = (wid + i * _NW) * _GR
            pltpu.async_copy(idx_hbm.at[pl.ds(g0, _GR)], idx_b[b], isem[b])

    def step(i, b):
        nb = 1 - b

        @pl.when(jnp.logical_and(i >= 3, i - 3 < ng))
        def _():
            pltpu.make_async_copy(
                out_b[nb], out_hbm.at[pl.ds(0, _GN), :], osem[nb]).wait()

        @pl.when(i < ng)
        def _():
            pltpu.make_async_copy(
                idx_hbm.at[pl.ds(0, _GR)], idx_b[b], isem[b]).wait()
            for k in range(_GR // 16):
                pltpu.async_copy(
                    tab_hbm.at[idx_b[b][pl.ds(k * 16, 16)]],
                    rows_b[b].at[pl.ds(k * 16, 16), :],
                    gsem[b])

        @pl.when(jnp.logical_and(i >= 1, i - 1 < ng))
        def _():
            pltpu.make_async_copy(
                tab_hbm.at[pl.ds(0, _GR), :], rows_b[nb], gsem[nb]).wait()
            for n in range(_GN):
                acc = tuple(
                    rows_b[nb][n * _NNZ, pl.ds(16 * j, 16)]
                    for j in range(_H // 16))

                def add_rows(m, a):
                    base = n * _NNZ + 1 + m * 7
                    for t in range(7):
                        a = tuple(
                            a[j] + rows_b[nb][base + t, pl.ds(16 * j, 16)]
                            for j in range(_H // 16))
                    return a

                acc = lax.fori_loop(0, 7, add_rows, acc)
                for j in range(_H // 16):
                    out_b[nb][n, pl.ds(16 * j, 16)] = jnp.maximum(
                        acc[j] + bias[j], 0.0)
            n0 = (wid + (i - 1) * _NW) * _GN
            pltpu.async_copy(out_b[nb], out_hbm.at[pl.ds(n0, _GN), :], osem[nb])

        stage(i + 1, nb)

    stage(0, 0)

    def pair(p, carry):
        step(2 * p, 0)
        step(2 * p + 1, 1)
        return carry

    lax.fori_loop(0, (ng + 4) // 2, pair, 0)


_ECH = 160
_EPT = _E // _NS
_NCH = _EPT // _ECH
_ZCH = 80
_NZCH = _N // _ZCH


@functools.partial(
    pl.kernel,
    out_type=jax.ShapeDtypeStruct((_NC, _N, _H), jnp.float32),
    mesh=_mesh,
    scratch_types=[
        pltpu.VMEM((_ECH,), jnp.int32),
        pltpu.VMEM((_ECH,), jnp.int32),
        pltpu.VMEM((_ECH,), jnp.int32),
        pltpu.VMEM((_ECH,), jnp.int32),
        pltpu.VMEM((_ECH,), jnp.float32),
        pltpu.VMEM((_ECH,), jnp.float32),
        pltpu.VMEM((_ECH, _H), jnp.float32),
        pltpu.VMEM((_ECH, _H), jnp.float32),
        pltpu.SemaphoreType.DMA,
        pltpu.SemaphoreType.DMA,
        pltpu.SemaphoreType.DMA,
        pltpu.SemaphoreType.DMA,
        pltpu.SemaphoreType.DMA,
        pltpu.SemaphoreType.DMA,
        pltpu.VMEM_SHARED((_N, _H), jnp.float32),
    ],
)
def _edges_sc(h_hbm, src_hbm, dst_hbm, w_hbm, z_hbm, out_hbm,
              sv0, sv1, dv0, dv1, wv0, wv1, rows0, rows1,
              isem0, isem1, gsem0, gsem1, ssem0, ssem1, acc):
    cid = lax.axis_index("c")
    sid = lax.axis_index("s")
    sv = (sv0, sv1)
    dv = (dv0, dv1)
    wv = (wv0, wv1)
    rows = (rows0, rows1)
    isem = (isem0, isem1)
    gsem = (gsem0, gsem1)
    ssem = (ssem0, ssem1)

    nz = (_NZCH - 1 - sid) // _NS + 1

    def zero_body(i, carry):
        r0 = (sid + i * _NS) * _ZCH
        pltpu.sync_copy(z_hbm.at[pl.ds(r0, _ZCH), :], acc.at[pl.ds(r0, _ZCH), :])
        return carry

    lax.fori_loop(0, nz, zero_body, 0)
    plsc.subcore_barrier()

    ebase = sid * _EPT

    def stage(i, b):
        @pl.when(i < _NCH)
        def _():
            e0 = ebase + i * _ECH
            pltpu.async_copy(src_hbm.at[pl.ds(e0, _ECH)], sv[b], isem[b])
            pltpu.async_copy(dst_hbm.at[pl.ds(e0, _ECH)], dv[b], isem[b])
            pltpu.async_copy(w_hbm.at[pl.ds(cid * _E + e0, _ECH)], wv[b], isem[b])

    def step(i, b):
        nb = 1 - b

        @pl.when(jnp.logical_and(i >= 2, i - 2 < _NCH))
        def _():
            pltpu.make_async_copy(
                rows[b], acc.at[pl.ds(0, _ECH), :], ssem[b]).wait()

        @pl.when(i < _NCH)
        def _():
            pltpu.make_async_copy(src_hbm.at[pl.ds(0, _ECH)], sv[b], isem[b]).wait()
            pltpu.make_async_copy(dst_hbm.at[pl.ds(0, _ECH)], dv[b], isem[b]).wait()
            pltpu.make_async_copy(w_hbm.at[pl.ds(0, _ECH)], wv[b], isem[b]).wait()
            for k in range(_ECH // 16):
                pltpu.async_copy(
                    h_hbm.at[sv[b][pl.ds(k * 16, 16)]],
                    rows[b].at[pl.ds(k * 16, 16), :],
                    gsem[b])

        @pl.when(jnp.logical_and(i >= 1, i - 1 < _NCH))
        def _():
            pltpu.make_async_copy(
                h_hbm.at[pl.ds(0, _ECH), :], rows[nb], gsem[nb]).wait()

            def scale_body(k, carry2):
                kb = k * 16
                w16 = wv[nb][pl.ds(kb, 16)]
                for e in range(16):
                    r = kb + e
                    wb = _bcast_lane(w16, e)
                    for j in range(_H // 16):
                        rows[nb][r, pl.ds(16 * j, 16)] = (
                            rows[nb][r, pl.ds(16 * j, 16)] * wb)
                return carry2

            lax.fori_loop(0, _ECH // 16, scale_body, 0)
            for k in range(_ECH // 16):
                pltpu.async_copy(
                    rows[nb].at[pl.ds(k * 16, 16), :],
                    acc.at[dv[nb][pl.ds(k * 16, 16)]],
                    ssem[nb], add=True)

        stage(i + 1, nb)

    stage(0, 0)

    def pair(p, carry):
        step(2 * p, 0)
        step(2 * p + 1, 1)
        return carry

    lax.fori_loop(0, (_NCH + 3) // 2 + 1, pair, 0)
    plsc.subcore_barrier()

    def wb_body(i, carry):
        r0 = (sid + i * _NS) * _ZCH
        pltpu.sync_copy(acc.at[pl.ds(r0, _ZCH), :],
                        out_hbm.at[cid, pl.ds(r0, _ZCH), :])
        return carry

    lax.fori_loop(0, nz, wb_body, 0)


_BLK = 1000


def _upd_body(p_ref, r_ref, w_ref, b_ref, o_ref):
    o_ref[...] = (
        jnp.maximum(
            jnp.dot(p_ref[...], w_ref[...], preferred_element_type=jnp.float32)
            + b_ref[...],
            0.0,
        )
        + r_ref[...]
    )


def _update_tc(ppi, res, W, b):
    return pl.pallas_call(
        _upd_body,
        grid=(_N // _BLK,),
        in_specs=[
            pl.BlockSpec((_BLK, _H), lambda i: (i, 0)),
            pl.BlockSpec((_BLK, _H), lambda i: (i, 0)),
            pl.BlockSpec((_H, _H), lambda i: (0, 0)),
            pl.BlockSpec((1, _H), lambda i: (0, 0)),
        ],
        out_specs=pl.BlockSpec((_BLK, _H), lambda i: (i, 0)),
        out_shape=jax.ShapeDtypeStruct((_N, _H), jnp.float32),
    )(ppi, res, W, b.reshape(1, _H))


def _fin_body(h_ref, w_ref, b_ref, o_ref):
    o_ref[...] = (
        jnp.dot(h_ref[...], w_ref[...], preferred_element_type=jnp.float32)
        + b_ref[...]
    )


def _final_tc(h, Wp, bp):
    return pl.pallas_call(
        _fin_body,
        grid=(_N // _BLK,),
        in_specs=[
            pl.BlockSpec((_BLK, _H), lambda i: (i, 0)),
            pl.BlockSpec((_H, _H), lambda i: (0, 0)),
            pl.BlockSpec((1, _H), lambda i: (0, 0)),
        ],
        out_specs=pl.BlockSpec((_BLK, _H), lambda i: (i, 0)),
        out_shape=jax.ShapeDtypeStruct((_N, _H), jnp.float32),
    )(h, Wp, bp.reshape(1, _H))


def kernel(indices, offsets, edge_index, ppi_w, self_w,
           emb_table, input_bias, W1, b1, W2, b2, Wout, bout):
    del offsets
    src = edge_index[0]
    dst = edge_index[1]
    w2 = jnp.concatenate([self_w, ppi_w])
    zeros = jnp.zeros((_N, _H), jnp.float32)
    h = _embed_sc(indices, emb_table, input_bias)
    for (W, b) in ((W1, b1), (W2, b2)):
        outs = _edges_sc(h, src, dst, w2, zeros)
        h = _update_tc(outs[1], outs[0], W, b)
    Wp = jnp.pad(Wout, ((0, 0), (0, _H - _L)))
    bp = jnp.pad(bout, (0, _H - _L))
    return _final_tc(h, Wp, bp)[:, :_L]
